# Initial kernel scaffold; baseline (speedup 1.0000x reference)
import jax
import jax.numpy as jnp
from jax.experimental import pallas as pl


def kernel(objectness, refine_loc, pred_conf, pred_loc, anchors, gt_boxes, gt_labels):
    raise NotImplementedError("write your pallas kernel here")


# speedup vs baseline: 5.3166x; 5.3166x over previous
import functools

import jax
import jax.numpy as jnp
from jax.experimental import pallas as pl
from jax.experimental.pallas import tpu as pltpu

_MATCH_THRESH = 0.5
_NEG_POS = 3
_V0 = 0.1
_V1 = 0.2
_A = 16320
_A_PAD = 16384
_R = 128
_L = 128
_G = 50
_C = 21


def _match_and_loss(pcx, pcy, pw, ph, logits, locpred, gtb_ref, gtl_ref,
                    use_labels, fiota, valid):
    px1 = pcx - pw * 0.5
    py1 = pcy - ph * 0.5
    px2 = pcx + pw * 0.5
    py2 = pcy + ph * 0.5
    wb = px2 - px1
    hb = py2 - py1
    area_b = wb * hb

    zero = jnp.zeros((_R, _L), jnp.float32)
    init = (zero, zero, zero, zero, zero,
            jnp.ones((_R, _L), jnp.int32))

    def g_body(g, carry):
        btv, m1, m2, m3, m4, lab = carry
        gx1 = gtb_ref[0, g, 0]
        gy1 = gtb_ref[0, g, 1]
        gx2 = gtb_ref[0, g, 2]
        gy2 = gtb_ref[0, g, 3]
        if use_labels:
            lg = gtl_ref[0, 0, g] + 1
        else:
            lg = 1
        ixmin = jnp.maximum(px1, gx1)
        iymin = jnp.maximum(py1, gy1)
        ixmax = jnp.minimum(px2, gx2)
        iymax = jnp.minimum(py2, gy2)
        iw = jnp.clip(ixmax - ixmin, 0.0, None)
        ih = jnp.clip(iymax - iymin, 0.0, None)
        inter = iw * ih
        area_a = (gx2 - gx1) * (gy2 - gy1)
        union = area_a + area_b - inter
        iou = inter / jnp.maximum(union, 1e-10)
        iou = jnp.where(valid, iou, 0.0)
        upd = iou > btv
        btv = jnp.where(upd, iou, btv)
        m1 = jnp.where(upd, gx1, m1)
        m2 = jnp.where(upd, gy1, m2)
        m3 = jnp.where(upd, gx2, m3)
        m4 = jnp.where(upd, gy2, m4)
        if use_labels:
            lab = jnp.where(upd, lg, lab)
        mval = jnp.max(iou)
        fidx = jnp.min(jnp.where(iou == mval, fiota, _A_PAD))
        fmask = fiota == fidx
        btv = jnp.where(fmask, 2.0, btv)
        m1 = jnp.where(fmask, gx1, m1)
        m2 = jnp.where(fmask, gy1, m2)
        m3 = jnp.where(fmask, gx2, m3)
        m4 = jnp.where(fmask, gy2, m4)
        if use_labels:
            lab = jnp.where(fmask, lg, lab)
        return (btv, m1, m2, m3, m4, lab)

    btv, m1, m2, m3, m4, lab = jax.lax.fori_loop(0, _G, g_body, init)

    conf = jnp.where(btv < _MATCH_THRESH, 0, lab)
    pos = conf > 0
    nposi = jnp.sum(pos.astype(jnp.int32))

    gcx = ((m1 + m3) * 0.5 - pcx) / (_V0 * pw)
    gcy = ((m2 + m4) * 0.5 - pcy) / (_V0 * ph)
    gw = jnp.log(jnp.maximum((m3 - m1) / pw, 1e-8)) / _V1
    gh = jnp.log(jnp.maximum((m4 - m2) / ph, 1e-8)) / _V1
    loc_loss = jnp.float32(0.0)
    for pred, tgt in zip(locpred, (gcx, gcy, gw, gh)):
        d = pred - tgt
        ad = jnp.abs(d)
        hub = jnp.where(ad < 1.0, 0.5 * d * d, ad - 0.5)
        loc_loss = loc_loss + jnp.sum(jnp.where(pos, hub, 0.0))

    mx = logits[0]
    for lg_ in logits[1:]:
        mx = jnp.maximum(mx, lg_)
    s = jnp.exp(logits[0] - mx)
    for lg_ in logits[1:]:
        s = s + jnp.exp(lg_ - mx)
    lse = mx + jnp.log(s)
    sel = logits[0]
    for c in range(1, len(logits)):
        sel = jnp.where(conf == c, logits[c], sel)
    ce = lse - sel

    pos_loss = jnp.sum(jnp.where(pos, ce, 0.0))

    neg = (conf == 0) & valid
    negcnt = jnp.sum(neg.astype(jnp.int32))
    negnum = jnp.maximum(10, jnp.minimum(nposi * _NEG_POS, _A - nposi))
    k = jnp.minimum(negnum, negcnt)
    x = jnp.where(neg, jax.lax.bitcast_convert_type(ce, jnp.int32),
                  jnp.int32(-1))

    def bs_body(_, lohi):
        lo, hi = lohi
        mid = lo + (hi - lo + 1) // 2
        cnt = jnp.sum((x >= mid).astype(jnp.int32))
        big = cnt >= k
        return (jnp.where(big, mid, lo), jnp.where(big, hi, mid - 1))

    lo, _hi = jax.lax.fori_loop(
        0, 31, bs_body, (jnp.int32(0), jnp.int32(0x7F800000)))
    v = jax.lax.bitcast_convert_type(lo, jnp.float32)
    gt_mask = x > lo
    cnt_gt = jnp.sum(gt_mask.astype(jnp.int32))
    sum_gt = jnp.sum(jnp.where(gt_mask, ce, 0.0))
    neg_loss = sum_gt + (k - cnt_gt).astype(jnp.float32) * v
    neg_loss = jnp.where(k > 0, neg_loss, 0.0)

    class_loss = pos_loss + neg_loss
    return class_loss, loc_loss, nposi.astype(jnp.float32)


def _body(obj_ref, rl_ref, pc_ref, plc_ref, an_ref, gtb_ref, gtl_ref,
          out_ref):
    b = pl.program_id(0)

    fiota = (jax.lax.broadcasted_iota(jnp.int32, (_R, _L), 0) * _L
             + jax.lax.broadcasted_iota(jnp.int32, (_R, _L), 1))
    valid = fiota < _A

    acx = an_ref[0]
    acy = an_ref[1]
    aw = an_ref[2]
    ah = an_ref[3]
    rl = [rl_ref[0, i] for i in range(4)]

    arm_cls, arm_loc, arm_n = _match_and_loss(
        acx, acy, aw, ah,
        [obj_ref[0, 0], obj_ref[0, 1]],
        rl, gtb_ref, gtl_ref, False, fiota, valid)

    ocx = acx + rl[0] * _V0 * aw
    ocy = acy + rl[1] * _V0 * ah
    ow = aw * jnp.exp(rl[2] * _V1)
    oh = ah * jnp.exp(rl[3] * _V1)
    odm_cls, odm_loc, odm_n = _match_and_loss(
        ocx, ocy, ow, oh,
        [pc_ref[0, c] for c in range(_C)],
        [plc_ref[0, i] for i in range(4)],
        gtb_ref, gtl_ref, True, fiota, valid)

    lane = jax.lax.broadcasted_iota(jnp.int32, (1, _L), 1)
    row = (jnp.where(lane == 0, arm_cls, 0.0)
           + jnp.where(lane == 1, arm_loc, 0.0)
           + jnp.where(lane == 2, arm_n, 0.0)
           + jnp.where(lane == 3, odm_cls, 0.0)
           + jnp.where(lane == 4, odm_loc, 0.0)
           + jnp.where(lane == 5, odm_n, 0.0))

    @pl.when(b == 0)
    def _():
        out_ref[...] = jnp.zeros((1, _L), jnp.float32)

    out_ref[...] += row


def _prep(x):
    xt = jnp.transpose(x, (0, 2, 1))
    xt = jnp.pad(xt, ((0, 0), (0, 0), (0, _A_PAD - _A)))
    return xt.reshape(x.shape[0], x.shape[2], _R, _L)


@jax.jit
def kernel(objectness, refine_loc, pred_conf, pred_loc, anchors, gt_boxes,
           gt_labels):
    B = objectness.shape[0]
    obj_t = _prep(objectness)
    rl_t = _prep(refine_loc)
    pc_t = _prep(pred_conf)
    plc_t = _prep(pred_loc)
    an_t = _prep(anchors[:1])[0]

    out = pl.pallas_call(
        _body,
        grid=(B,),
        in_specs=[
            pl.BlockSpec((1, 2, _R, _L), lambda b: (b, 0, 0, 0)),
            pl.BlockSpec((1, 4, _R, _L), lambda b: (b, 0, 0, 0)),
            pl.BlockSpec((1, _C, _R, _L), lambda b: (b, 0, 0, 0)),
            pl.BlockSpec((1, 4, _R, _L), lambda b: (b, 0, 0, 0)),
            pl.BlockSpec((4, _R, _L), lambda b: (0, 0, 0)),
            pl.BlockSpec((1, _G, 4), lambda b: (b, 0, 0),
                         memory_space=pltpu.SMEM),
            pl.BlockSpec((1, 1, _G), lambda b: (b, 0, 0),
                         memory_space=pltpu.SMEM),
        ],
        out_specs=pl.BlockSpec((1, _L), lambda b: (0, 0)),
        out_shape=jax.ShapeDtypeStruct((1, _L), jnp.float32),
    )(obj_t, rl_t, pc_t, plc_t, an_t, gt_boxes,
      gt_labels.reshape(B, 1, _G))

    r = out[0]
    arm_cls = r[0] / r[2]
    arm_loc = r[1] / r[2]
    odm_cls = r[3] / r[5]
    odm_loc = r[4] / r[5]
    total = arm_cls + arm_loc + odm_cls + odm_loc
    return (total, odm_cls, odm_loc, arm_cls, arm_loc)


# speedup vs baseline: 8.2574x; 1.5531x over previous
import functools

import jax
import jax.numpy as jnp
from jax.experimental import pallas as pl
from jax.experimental.pallas import tpu as pltpu

_MATCH_THRESH = 0.5
_NEG_POS = 3
_V0 = 0.1
_V1 = 0.2
_A = 16320
_A_PAD = 16384
_R = 128
_L = 128
_G = 50
_C = 21


def _match_and_loss(pcx, pcy, pw, ph, logits, locpred, gtb_ref, gtl_ref,
                    use_labels, fiota, valid):
    px1 = pcx - pw * 0.5
    py1 = pcy - ph * 0.5
    px2 = pcx + pw * 0.5
    py2 = pcy + ph * 0.5
    wb = px2 - px1
    hb = py2 - py1
    area_b = wb * hb

    zero = jnp.zeros((_R, _L), jnp.float32)
    btv = zero
    bg = jnp.zeros((_R, _L), jnp.int32)

    rowidx = []
    for g in range(_G):
        gx1 = gtb_ref[0, g, 0]
        gy1 = gtb_ref[0, g, 1]
        gx2 = gtb_ref[0, g, 2]
        gy2 = gtb_ref[0, g, 3]
        ixmin = jnp.maximum(px1, gx1)
        iymin = jnp.maximum(py1, gy1)
        ixmax = jnp.minimum(px2, gx2)
        iymax = jnp.minimum(py2, gy2)
        iw = jnp.clip(ixmax - ixmin, 0.0, None)
        ih = jnp.clip(iymax - iymin, 0.0, None)
        inter = iw * ih
        area_a = (gx2 - gx1) * (gy2 - gy1)
        union = area_a + area_b - inter
        iou = inter / jnp.maximum(union, 1e-10)
        iou = jnp.where(valid, iou, 0.0)
        upd = iou > btv
        btv = jnp.where(upd, iou, btv)
        bg = jnp.where(upd, g, bg)
        mval = jnp.max(iou)
        rowidx.append(jnp.min(jnp.where(iou == mval, fiota, _A_PAD)))

    for g in range(_G):
        fmask = fiota == rowidx[g]
        btv = jnp.where(fmask, 2.0, btv)
        bg = jnp.where(fmask, g, bg)

    m1 = zero
    m2 = zero
    m3 = zero
    m4 = zero
    lab = jnp.ones((_R, _L), jnp.int32)
    for g in range(_G):
        eqg = bg == g
        m1 = jnp.where(eqg, gtb_ref[0, g, 0], m1)
        m2 = jnp.where(eqg, gtb_ref[0, g, 1], m2)
        m3 = jnp.where(eqg, gtb_ref[0, g, 2], m3)
        m4 = jnp.where(eqg, gtb_ref[0, g, 3], m4)
        if use_labels:
            lab = jnp.where(eqg, gtl_ref[0, 0, g] + 1, lab)

    conf = jnp.where(btv < _MATCH_THRESH, 0, lab)
    pos = conf > 0
    nposi = jnp.sum(pos.astype(jnp.int32))

    gcx = ((m1 + m3) * 0.5 - pcx) / (_V0 * pw)
    gcy = ((m2 + m4) * 0.5 - pcy) / (_V0 * ph)
    gw = jnp.log(jnp.maximum((m3 - m1) / pw, 1e-8)) / _V1
    gh = jnp.log(jnp.maximum((m4 - m2) / ph, 1e-8)) / _V1
    loc_loss = jnp.float32(0.0)
    for pred, tgt in zip(locpred, (gcx, gcy, gw, gh)):
        d = pred - tgt
        ad = jnp.abs(d)
        hub = jnp.where(ad < 1.0, 0.5 * d * d, ad - 0.5)
        loc_loss = loc_loss + jnp.sum(jnp.where(pos, hub, 0.0))

    mx = logits[0]
    for lg_ in logits[1:]:
        mx = jnp.maximum(mx, lg_)
    s = jnp.exp(logits[0] - mx)
    for lg_ in logits[1:]:
        s = s + jnp.exp(lg_ - mx)
    lse = mx + jnp.log(s)
    sel = logits[0]
    for c in range(1, len(logits)):
        sel = jnp.where(conf == c, logits[c], sel)
    ce = lse - sel

    pos_loss = jnp.sum(jnp.where(pos, ce, 0.0))

    neg = (conf == 0) & valid
    negcnt = jnp.sum(neg.astype(jnp.int32))
    negnum = jnp.maximum(10, jnp.minimum(nposi * _NEG_POS, _A - nposi))
    k = jnp.minimum(negnum, negcnt)
    x = jnp.where(neg, jax.lax.bitcast_convert_type(ce, jnp.int32),
                  jnp.int32(-1))

    def bs_body(_, lohi):
        lo, hi = lohi
        mid = lo + (hi - lo + 1) // 2
        cnt = jnp.sum((x >= mid).astype(jnp.int32))
        big = cnt >= k
        return (jnp.where(big, mid, lo), jnp.where(big, hi, mid - 1))

    lo, _hi = jax.lax.fori_loop(
        0, 31, bs_body, (jnp.int32(0), jnp.int32(0x7F800000)))
    v = jax.lax.bitcast_convert_type(lo, jnp.float32)
    gt_mask = x > lo
    cnt_gt = jnp.sum(gt_mask.astype(jnp.int32))
    sum_gt = jnp.sum(jnp.where(gt_mask, ce, 0.0))
    neg_loss = sum_gt + (k - cnt_gt).astype(jnp.float32) * v
    neg_loss = jnp.where(k > 0, neg_loss, 0.0)

    class_loss = pos_loss + neg_loss
    return class_loss, loc_loss, nposi.astype(jnp.float32)


def _body(obj_ref, rl_ref, pc_ref, plc_ref, an_ref, gtb_ref, gtl_ref,
          out_ref):
    b = pl.program_id(0)

    fiota = (jax.lax.broadcasted_iota(jnp.int32, (_R, _L), 0) * _L
             + jax.lax.broadcasted_iota(jnp.int32, (_R, _L), 1))
    valid = fiota < _A

    acx = an_ref[0]
    acy = an_ref[1]
    aw = an_ref[2]
    ah = an_ref[3]
    rl = [rl_ref[0, i] for i in range(4)]

    arm_cls, arm_loc, arm_n = _match_and_loss(
        acx, acy, aw, ah,
        [obj_ref[0, 0], obj_ref[0, 1]],
        rl, gtb_ref, gtl_ref, False, fiota, valid)

    ocx = acx + rl[0] * _V0 * aw
    ocy = acy + rl[1] * _V0 * ah
    ow = aw * jnp.exp(rl[2] * _V1)
    oh = ah * jnp.exp(rl[3] * _V1)
    odm_cls, odm_loc, odm_n = _match_and_loss(
        ocx, ocy, ow, oh,
        [pc_ref[0, c] for c in range(_C)],
        [plc_ref[0, i] for i in range(4)],
        gtb_ref, gtl_ref, True, fiota, valid)

    lane = jax.lax.broadcasted_iota(jnp.int32, (1, _L), 1)
    row = (jnp.where(lane == 0, arm_cls, 0.0)
           + jnp.where(lane == 1, arm_loc, 0.0)
           + jnp.where(lane == 2, arm_n, 0.0)
           + jnp.where(lane == 3, odm_cls, 0.0)
           + jnp.where(lane == 4, odm_loc, 0.0)
           + jnp.where(lane == 5, odm_n, 0.0))

    @pl.when(b == 0)
    def _():
        out_ref[...] = jnp.zeros((1, _L), jnp.float32)

    out_ref[...] += row


def _prep(x):
    xt = jnp.transpose(x, (0, 2, 1))
    xt = jnp.pad(xt, ((0, 0), (0, 0), (0, _A_PAD - _A)))
    return xt.reshape(x.shape[0], x.shape[2], _R, _L)


@jax.jit
def kernel(objectness, refine_loc, pred_conf, pred_loc, anchors, gt_boxes,
           gt_labels):
    B = objectness.shape[0]
    obj_t = _prep(objectness)
    rl_t = _prep(refine_loc)
    pc_t = _prep(pred_conf)
    plc_t = _prep(pred_loc)
    an_t = _prep(anchors[:1])[0]

    out = pl.pallas_call(
        _body,
        grid=(B,),
        in_specs=[
            pl.BlockSpec((1, 2, _R, _L), lambda b: (b, 0, 0, 0)),
            pl.BlockSpec((1, 4, _R, _L), lambda b: (b, 0, 0, 0)),
            pl.BlockSpec((1, _C, _R, _L), lambda b: (b, 0, 0, 0)),
            pl.BlockSpec((1, 4, _R, _L), lambda b: (b, 0, 0, 0)),
            pl.BlockSpec((4, _R, _L), lambda b: (0, 0, 0)),
            pl.BlockSpec((1, _G, 4), lambda b: (b, 0, 0),
                         memory_space=pltpu.SMEM),
            pl.BlockSpec((1, 1, _G), lambda b: (b, 0, 0),
                         memory_space=pltpu.SMEM),
        ],
        out_specs=pl.BlockSpec((1, _L), lambda b: (0, 0)),
        out_shape=jax.ShapeDtypeStruct((1, _L), jnp.float32),
    )(obj_t, rl_t, pc_t, plc_t, an_t, gt_boxes,
      gt_labels.reshape(B, 1, _G))

    r = out[0]
    arm_cls = r[0] / r[2]
    arm_loc = r[1] / r[2]
    odm_cls = r[3] / r[5]
    odm_loc = r[4] / r[5]
    total = arm_cls + arm_loc + odm_cls + odm_loc
    return (total, odm_cls, odm_loc, arm_cls, arm_loc)
